# speedup vs baseline: 31.0535x; 31.0535x over previous
import functools

import jax
import jax.numpy as jnp
from jax import lax
from jax.experimental import pallas as pl
from jax.experimental.pallas import tpu as pltpu

_F32 = jnp.float32


def _top3_axis0(dist, S):
    iota = lax.broadcasted_iota(jnp.int32, dist.shape, 0)
    inf = jnp.array(jnp.inf, _F32)
    m1 = jnp.min(dist, axis=0, keepdims=True)
    i1 = jnp.min(jnp.where(dist == m1, iota, S), axis=0, keepdims=True)
    d1 = jnp.where(iota == i1, inf, dist)
    m2 = jnp.min(d1, axis=0, keepdims=True)
    i2 = jnp.min(jnp.where(d1 == m2, iota, S), axis=0, keepdims=True)
    d2 = jnp.where(iota == i2, inf, d1)
    m3 = jnp.min(d2, axis=0, keepdims=True)
    i3 = jnp.min(jnp.where(d2 == m3, iota, S), axis=0, keepdims=True)
    return (m1, m2, m3), (i1, i2, i3)


def _inv_dist_weights(m1, m2, m3):
    d1 = jnp.maximum(m1, 1e-8)
    d2 = jnp.maximum(m2, 1e-8)
    d3 = jnp.maximum(m3, 1e-8)
    w1 = 1.0 / d1
    w2 = 1.0 / d2
    w3 = 1.0 / d3
    s = w1 + w2 + w3
    return w1 / s, w2 / s, w3 / s


def _onehot_t(i123, w123, S, T):
    iota = lax.broadcasted_iota(jnp.int32, (S, T), 0)
    zero = jnp.array(0.0, _F32)
    (i1, i2, i3), (w1, w2, w3) = i123, w123
    return (jnp.where(iota == i1, w1, zero)
            + jnp.where(iota == i2, w2, zero)
            + jnp.where(iota == i3, w3, zero))


def _dist_matrix(src, dstT):
    cross = jnp.dot(src, dstT, preferred_element_type=_F32)
    s2 = jnp.sum(src * src, axis=1, keepdims=True)
    t2 = jnp.sum(dstT * dstT, axis=0, keepdims=True)
    d2 = s2 + t2 - 2.0 * cross
    return jnp.sqrt(jnp.maximum(d2, 0.0))


def _interp_t(ohT, g):
    return lax.dot_general(ohT, g, (((0,), (0,)), ((), ())),
                           preferred_element_type=_F32)


def _a1_body(f1, f2, f3, c2, c3, c1T, c2T,
             w3at, w3ab, b3a, w3b, b3b,
             w2at, w2ab, b2a, w2b, b2b, w1a, g1_out):
    f1v, f2v, f3v = f1[0], f2[0], f3[0]
    c2v, c3v = c2[0], c3[0]
    c1Tv, c2Tv = c1T[0], c2T[0]
    S3, S2, S1 = f3v.shape[0], f2v.shape[0], f1v.shape[0]

    dist = _dist_matrix(c3v, c2Tv)
    ms, is_ = _top3_axis0(dist, S3)
    ws = _inv_dist_weights(*ms)
    ohT = _onehot_t(is_, ws, S3, S2)
    g3 = jnp.dot(f3v, w3ab[...], preferred_element_type=_F32)
    up = _interp_t(ohT, g3)
    skip = jnp.dot(f2v, w3at[...], preferred_element_type=_F32)
    h = jnp.maximum(skip + up + b3a[...], 0.0)
    fused2 = jnp.dot(h, w3b[...], preferred_element_type=_F32) + b3b[...]

    dist = _dist_matrix(c2v, c1Tv)
    ms, is_ = _top3_axis0(dist, S2)
    ws = _inv_dist_weights(*ms)
    ohT = _onehot_t(is_, ws, S2, S1)
    g2 = jnp.dot(fused2, w2ab[...], preferred_element_type=_F32)
    up = _interp_t(ohT, g2)
    skip = jnp.dot(f1v, w2at[...], preferred_element_type=_F32)
    h = jnp.maximum(skip + up + b2a[...], 0.0)
    fused1 = jnp.dot(h, w2b[...], preferred_element_type=_F32) + b2b[...]

    g1_out[0] = jnp.dot(fused1, w1a[...], preferred_element_type=_F32)


def _a2_body(c1, xT, idx_out, w_out, *, S):
    c1v = c1[0]
    xTv = xT[0]
    blk = xTv.shape[1]
    cross = jnp.dot(c1v, xTv, preferred_element_type=_F32)
    s2 = jnp.sum(c1v * c1v, axis=1, keepdims=True)
    t2 = jnp.sum(xTv * xTv, axis=0, keepdims=True)
    dist = jnp.sqrt(jnp.maximum(s2 + t2 - 2.0 * cross, 0.0))
    ms, (i1, i2, i3) = _top3_axis0(dist, S)
    w1, w2, w3 = _inv_dist_weights(*ms)
    zi = jnp.zeros((5, blk), jnp.int32)
    zf = jnp.zeros((5, blk), _F32)
    idx_out[0] = jnp.concatenate([i1, i2, i3, zi], axis=0)
    w_out[0] = jnp.concatenate([w1, w2, w3, zf], axis=0)


def _b_body(g1, idx8, w8, b1a, w1b, b1b, out, *, S):
    g = g1[0]
    idx = idx8[0]
    w = w8[0]
    blk = idx.shape[1]
    i123 = (idx[0:1], idx[1:2], idx[2:3])
    w123 = (w[0:1], w[1:2], w[2:3])
    ohT = _onehot_t(i123, w123, S, blk)
    up = _interp_t(ohT, g)
    h = jnp.maximum(up + b1a[...], 0.0)
    out[0] = jnp.dot(h, w1b[...], preferred_element_type=_F32) + b1b[...]


def kernel(feat1, feat2, feat3, ctr1, ctr2, ctr3, xyz,
           w3a, b3a, w3b, b3b, w2a, b2a, w2b, b2b, w1a, b1a, w1b, b1b):
    B, N, C = feat1.shape[0], xyz.shape[1], feat1.shape[2]
    S1, S2, S3 = feat1.shape[1], feat2.shape[1], feat3.shape[1]

    ctr1T = jnp.swapaxes(ctr1, 1, 2)
    ctr2T = jnp.swapaxes(ctr2, 1, 2)
    xyzT = jnp.swapaxes(xyz, 1, 2)
    w3at, w3ab = w3a[:C], w3a[C:]
    w2at, w2ab = w2a[:C], w2a[C:]
    b3a2 = b3a.reshape(1, C)
    b3b2 = b3b.reshape(1, C)
    b2a2 = b2a.reshape(1, C)
    b2b2 = b2b.reshape(1, C)
    b1a2 = b1a.reshape(1, C)
    b1b2 = b1b.reshape(1, C)

    full = lambda shape: pl.BlockSpec(shape, lambda *_: (0,) * len(shape))
    batch = lambda shape: pl.BlockSpec((1,) + shape,
                                       lambda b, *_: (b,) + (0,) * len(shape))

    g1 = pl.pallas_call(
        _a1_body,
        grid=(B,),
        in_specs=[
            batch((S1, C)), batch((S2, C)), batch((S3, C)),
            batch((S2, 3)), batch((S3, 3)),
            batch((3, S1)), batch((3, S2)),
            full((C, C)), full((C, C)), full((1, C)), full((C, C)), full((1, C)),
            full((C, C)), full((C, C)), full((1, C)), full((C, C)), full((1, C)),
            full((C, C)),
        ],
        out_specs=batch((S1, C)),
        out_shape=jax.ShapeDtypeStruct((B, S1, C), _F32),
    )(feat1, feat2, feat3, ctr2, ctr3, ctr1T, ctr2T,
      w3at, w3ab, b3a2, w3b, b3b2, w2at, w2ab, b2a2, w2b, b2b2, w1a)

    BLK_A = 1024
    grid_a = (B, N // BLK_A)
    idx8, w8 = pl.pallas_call(
        functools.partial(_a2_body, S=S1),
        grid=grid_a,
        in_specs=[
            pl.BlockSpec((1, S1, 3), lambda b, n: (b, 0, 0)),
            pl.BlockSpec((1, 3, BLK_A), lambda b, n: (b, 0, n)),
        ],
        out_specs=[
            pl.BlockSpec((1, 8, BLK_A), lambda b, n: (b, 0, n)),
            pl.BlockSpec((1, 8, BLK_A), lambda b, n: (b, 0, n)),
        ],
        out_shape=[
            jax.ShapeDtypeStruct((B, 8, N), jnp.int32),
            jax.ShapeDtypeStruct((B, 8, N), _F32),
        ],
    )(ctr1, xyzT)

    BLK_B = 1024
    grid_b = (B, N // BLK_B)
    out = pl.pallas_call(
        functools.partial(_b_body, S=S1),
        grid=grid_b,
        in_specs=[
            pl.BlockSpec((1, S1, C), lambda b, n: (b, 0, 0)),
            pl.BlockSpec((1, 8, BLK_B), lambda b, n: (b, 0, n)),
            pl.BlockSpec((1, 8, BLK_B), lambda b, n: (b, 0, n)),
            pl.BlockSpec((1, C), lambda b, n: (0, 0)),
            pl.BlockSpec((C, C), lambda b, n: (0, 0)),
            pl.BlockSpec((1, C), lambda b, n: (0, 0)),
        ],
        out_specs=pl.BlockSpec((1, BLK_B, C), lambda b, n: (b, n, 0)),
        out_shape=jax.ShapeDtypeStruct((B, N, C), _F32),
    )(g1, idx8, w8, b1a2, w1b, b1b2)
    return out


# speedup vs baseline: 36.0479x; 1.1608x over previous
import functools

import jax
import jax.numpy as jnp
from jax import lax
from jax.experimental import pallas as pl
from jax.experimental.pallas import tpu as pltpu

_F32 = jnp.float32


def _top3_axis0(dist, S):
    iota = lax.broadcasted_iota(jnp.int32, dist.shape, 0)
    inf = jnp.array(jnp.inf, _F32)
    m1 = jnp.min(dist, axis=0, keepdims=True)
    i1 = jnp.min(jnp.where(dist == m1, iota, S), axis=0, keepdims=True)
    d1 = jnp.where(iota == i1, inf, dist)
    m2 = jnp.min(d1, axis=0, keepdims=True)
    i2 = jnp.min(jnp.where(d1 == m2, iota, S), axis=0, keepdims=True)
    d2 = jnp.where(iota == i2, inf, d1)
    m3 = jnp.min(d2, axis=0, keepdims=True)
    i3 = jnp.min(jnp.where(d2 == m3, iota, S), axis=0, keepdims=True)
    return (m1, m2, m3), (i1, i2, i3)


def _inv_dist_weights(m1, m2, m3):
    d1 = jnp.maximum(m1, 1e-8)
    d2 = jnp.maximum(m2, 1e-8)
    d3 = jnp.maximum(m3, 1e-8)
    w1 = 1.0 / d1
    w2 = 1.0 / d2
    w3 = 1.0 / d3
    s = w1 + w2 + w3
    return w1 / s, w2 / s, w3 / s


def _onehot_t(i123, w123, S, T):
    iota = lax.broadcasted_iota(jnp.int32, (S, T), 0)
    zero = jnp.array(0.0, _F32)
    (i1, i2, i3), (w1, w2, w3) = i123, w123
    return (jnp.where(iota == i1, w1, zero)
            + jnp.where(iota == i2, w2, zero)
            + jnp.where(iota == i3, w3, zero))


def _dist_matrix(src, dstT):
    cross = jnp.dot(src, dstT, preferred_element_type=_F32)
    s2 = jnp.sum(src * src, axis=1, keepdims=True)
    t2 = jnp.sum(dstT * dstT, axis=0, keepdims=True)
    d2 = s2 + t2 - 2.0 * cross
    return jnp.sqrt(jnp.maximum(d2, 0.0))


def _interp_t(ohT, g):
    return lax.dot_general(ohT, g, (((0,), (0,)), ((), ())),
                           preferred_element_type=_F32)


def _a1_body(f1, f2, f3, c2, c3, c1T, c2T,
             w3at, w3ab, b3a, w3b, b3b,
             w2at, w2ab, b2a, w2b, b2b, w1a, g1_out):
    f1v, f2v, f3v = f1[0], f2[0], f3[0]
    c2v, c3v = c2[0], c3[0]
    c1Tv, c2Tv = c1T[0], c2T[0]
    S3, S2, S1 = f3v.shape[0], f2v.shape[0], f1v.shape[0]

    dist = _dist_matrix(c3v, c2Tv)
    ms, is_ = _top3_axis0(dist, S3)
    ws = _inv_dist_weights(*ms)
    ohT = _onehot_t(is_, ws, S3, S2)
    g3 = jnp.dot(f3v, w3ab[...], preferred_element_type=_F32)
    up = _interp_t(ohT, g3)
    skip = jnp.dot(f2v, w3at[...], preferred_element_type=_F32)
    h = jnp.maximum(skip + up + b3a[...], 0.0)
    fused2 = jnp.dot(h, w3b[...], preferred_element_type=_F32) + b3b[...]

    dist = _dist_matrix(c2v, c1Tv)
    ms, is_ = _top3_axis0(dist, S2)
    ws = _inv_dist_weights(*ms)
    ohT = _onehot_t(is_, ws, S2, S1)
    g2 = jnp.dot(fused2, w2ab[...], preferred_element_type=_F32)
    up = _interp_t(ohT, g2)
    skip = jnp.dot(f1v, w2at[...], preferred_element_type=_F32)
    h = jnp.maximum(skip + up + b2a[...], 0.0)
    fused1 = jnp.dot(h, w2b[...], preferred_element_type=_F32) + b2b[...]

    g1_out[0] = jnp.dot(fused1, w1a[...], preferred_element_type=_F32)


def _a2b_body(c1, xT, g1, b1a, w1b, b1b, out, *, S):
    c1v = c1[0]
    xTv = xT[0]
    g = g1[0]
    blk = xTv.shape[1]
    c1sq = jnp.sum(c1v * c1v, axis=1, keepdims=True)
    lhs4 = jnp.concatenate([c1v, c1sq], axis=1)
    rhs4 = jnp.concatenate([-2.0 * xTv, jnp.ones((1, blk), _F32)], axis=0)
    s = jnp.dot(lhs4, rhs4, preferred_element_type=_F32)
    (m1, m2, m3), i123 = _top3_axis0(s, S)
    x2 = jnp.sum(xTv * xTv, axis=0, keepdims=True)
    d1 = jnp.sqrt(jnp.maximum(m1 + x2, 0.0))
    d2 = jnp.sqrt(jnp.maximum(m2 + x2, 0.0))
    d3 = jnp.sqrt(jnp.maximum(m3 + x2, 0.0))
    w123 = _inv_dist_weights(d1, d2, d3)
    ohT = _onehot_t(i123, w123, S, blk)
    up = _interp_t(ohT, g)
    h = jnp.maximum(up + b1a[...], 0.0)
    out[0] = jnp.dot(h, w1b[...], preferred_element_type=_F32) + b1b[...]


def kernel(feat1, feat2, feat3, ctr1, ctr2, ctr3, xyz,
           w3a, b3a, w3b, b3b, w2a, b2a, w2b, b2b, w1a, b1a, w1b, b1b):
    B, N, C = feat1.shape[0], xyz.shape[1], feat1.shape[2]
    S1, S2, S3 = feat1.shape[1], feat2.shape[1], feat3.shape[1]

    ctr1T = jnp.swapaxes(ctr1, 1, 2)
    ctr2T = jnp.swapaxes(ctr2, 1, 2)
    xyzT = jnp.swapaxes(xyz, 1, 2)
    w3at, w3ab = w3a[:C], w3a[C:]
    w2at, w2ab = w2a[:C], w2a[C:]
    b3a2 = b3a.reshape(1, C)
    b3b2 = b3b.reshape(1, C)
    b2a2 = b2a.reshape(1, C)
    b2b2 = b2b.reshape(1, C)
    b1a2 = b1a.reshape(1, C)
    b1b2 = b1b.reshape(1, C)

    full = lambda shape: pl.BlockSpec(shape, lambda *_: (0,) * len(shape))
    batch = lambda shape: pl.BlockSpec((1,) + shape,
                                       lambda b, *_: (b,) + (0,) * len(shape))

    g1 = pl.pallas_call(
        _a1_body,
        grid=(B,),
        in_specs=[
            batch((S1, C)), batch((S2, C)), batch((S3, C)),
            batch((S2, 3)), batch((S3, 3)),
            batch((3, S1)), batch((3, S2)),
            full((C, C)), full((C, C)), full((1, C)), full((C, C)), full((1, C)),
            full((C, C)), full((C, C)), full((1, C)), full((C, C)), full((1, C)),
            full((C, C)),
        ],
        out_specs=batch((S1, C)),
        out_shape=jax.ShapeDtypeStruct((B, S1, C), _F32),
    )(feat1, feat2, feat3, ctr2, ctr3, ctr1T, ctr2T,
      w3at, w3ab, b3a2, w3b, b3b2, w2at, w2ab, b2a2, w2b, b2b2, w1a)

    BLK = 1024
    grid_b = (B, N // BLK)
    out = pl.pallas_call(
        functools.partial(_a2b_body, S=S1),
        grid=grid_b,
        in_specs=[
            pl.BlockSpec((1, S1, 3), lambda b, n: (b, 0, 0)),
            pl.BlockSpec((1, 3, BLK), lambda b, n: (b, 0, n)),
            pl.BlockSpec((1, S1, C), lambda b, n: (b, 0, 0)),
            pl.BlockSpec((1, C), lambda b, n: (0, 0)),
            pl.BlockSpec((C, C), lambda b, n: (0, 0)),
            pl.BlockSpec((1, C), lambda b, n: (0, 0)),
        ],
        out_specs=pl.BlockSpec((1, BLK, C), lambda b, n: (b, n, 0)),
        out_shape=jax.ShapeDtypeStruct((B, N, C), _F32),
    )(ctr1, xyzT, g1, b1a2, w1b, b1b2)
    return out
